# baseline (speedup 1.0000x reference)
import functools

import jax
import jax.numpy as jnp
from jax import lax
from jax.experimental import pallas as pl
from jax.experimental.pallas import tpu as pltpu
from jax.experimental.pallas import tpu_sc as plsc

D_MODEL = 64
VOCAB = 1000000
BATCH = 4096
HIST = 50

NC = 2
NS = 16
NW = NC * NS

GRP = BATCH // NW
N_STAGE = HIST

SCALE = 8.0


def _mesh():
    return plsc.VectorSubcoreMesh(core_axis_name="c", subcore_axis_name="s")


@functools.partial(
    pl.kernel,
    mesh=_mesh(),
    out_type=jax.ShapeDtypeStruct((HIST, D_MODEL, BATCH), jnp.float32),
    scratch_types=[
        pltpu.VMEM((N_STAGE, GRP), jnp.int32),
        pltpu.VMEM((2, GRP // 8, 8, D_MODEL), jnp.float32),
        pltpu.VMEM((2, D_MODEL, GRP), jnp.float32),
        pltpu.SemaphoreType.DMA,
        pltpu.SemaphoreType.DMA,
    ],
    compiler_params=pltpu.CompilerParams(needs_layout_passes=False),
)
def _gather_scale(idx_hbm, table_hbm, out_hbm, idx_v, buf, obuf, sem0, sem1):
    wid = lax.axis_index("s") * NC + lax.axis_index("c")
    b0 = pl.multiple_of(wid * GRP, GRP)
    sems = (sem0, sem1)
    pltpu.sync_copy(idx_hbm.at[:, pl.ds(b0, GRP)], idx_v)

    def fire(st, p):
        def c_body(c, carry):
            chunk = idx_v[st, pl.ds(c * 16, 16)]
            for i in range(16):
                v = chunk[i]
                pltpu.async_copy(
                    table_hbm.at[lax.shift_right_logical(v, 3),
                                 lax.bitwise_and(v, 7)],
                    buf.at[p, 2 * c + i // 8, i % 8],
                    sems[p])
            return carry
        lax.fori_loop(0, GRP // 16, c_body, 0)

    def drain(p):
        pltpu.make_async_copy(
            table_hbm.at[pl.ds(0, GRP // 8)], buf.at[p], sems[p]).wait()

    qrows = [lax.iota(jnp.int32, 16) + q * 16 for q in range(D_MODEL // 16)]

    def scale_store(st, p):
        def r_body(r, carry):
            t0 = r * 8
            for j in range(8):
                tcol = jnp.broadcast_to(t0 + j, (16,))
                for q in range(D_MODEL // 16):
                    sl = pl.ds(q * 16, 16)
                    v = buf[p, r, j, sl] * SCALE
                    plsc.store_scatter(obuf.at[p], [qrows[q], tcol], v)
            return carry
        lax.fori_loop(0, GRP // 8, r_body, 0)
        pltpu.sync_copy(obuf.at[p], out_hbm.at[st, :, pl.ds(b0, GRP)])

    fire(0, 0)

    def pair_body(u, carry):
        for q in range(2):
            st = 2 * u + q
            @pl.when(st + 1 < N_STAGE)
            def _fire_next():
                fire(st + 1, 1 - q)
            drain(q)
            scale_store(st, q)
        return carry
    lax.fori_loop(0, N_STAGE // 2, pair_body, 0)


def kernel(x, W):
    idx = jnp.transpose(x.reshape(BATCH, HIST)).astype(jnp.int32)
    table = W.reshape(VOCAB // 8, 8, D_MODEL)
    out = _gather_scale(idx, table)
    return jnp.transpose(out, (2, 0, 1))


# speedup vs baseline: 1.5716x; 1.5716x over previous
import functools

import jax
import jax.numpy as jnp
from jax import lax
from jax.experimental import pallas as pl
from jax.experimental.pallas import tpu as pltpu
from jax.experimental.pallas import tpu_sc as plsc

D_MODEL = 64
VOCAB = 1000000
BATCH = 4096
HIST = 50

NC = 2
NS = 16
NW = NC * NS

GRP = BATCH // NW
N_STAGE = HIST

SCALE = 8.0


def _mesh():
    return plsc.VectorSubcoreMesh(core_axis_name="c", subcore_axis_name="s")


@functools.partial(
    pl.kernel,
    mesh=_mesh(),
    out_type=jax.ShapeDtypeStruct((HIST, NW, GRP // 8, 8, D_MODEL),
                                  jnp.float32),
    scratch_types=[
        pltpu.VMEM((N_STAGE, GRP), jnp.int32),
        pltpu.VMEM((2, GRP // 8, 8, D_MODEL), jnp.float32),
        pltpu.SemaphoreType.DMA,
        pltpu.SemaphoreType.DMA,
        pltpu.SemaphoreType.DMA,
        pltpu.SemaphoreType.DMA,
    ],
    compiler_params=pltpu.CompilerParams(needs_layout_passes=False),
)
def _gather_scale(idx_hbm, table_hbm, out_hbm, idx_v, buf, sem0, sem1,
                  osem0, osem1):
    wid = lax.axis_index("s") * NC + lax.axis_index("c")
    b0 = pl.multiple_of(wid * GRP, GRP)
    sems = (sem0, sem1)
    osems = (osem0, osem1)
    pltpu.sync_copy(idx_hbm.at[:, pl.ds(b0, GRP)], idx_v)

    def store_drain(st, p):
        pltpu.make_async_copy(
            buf.at[p], out_hbm.at[st, wid], osems[p]).wait()

    def fire(st, p):
        def c_body(c, carry):
            chunk = idx_v[st, pl.ds(c * 16, 16)]
            for i in range(16):
                v = chunk[i]
                pltpu.async_copy(
                    table_hbm.at[lax.shift_right_logical(v, 3),
                                 lax.bitwise_and(v, 7)],
                    buf.at[p, 2 * c + i // 8, i % 8],
                    sems[p])
            return carry
        lax.fori_loop(0, GRP // 16, c_body, 0)

    def drain(p):
        pltpu.make_async_copy(
            table_hbm.at[pl.ds(0, GRP // 8)], buf.at[p], sems[p]).wait()

    def scale_store(st, p):
        def r_body(r, carry):
            for j in range(8):
                for q in range(D_MODEL // 16):
                    sl = pl.ds(q * 16, 16)
                    buf[p, r, j, sl] = buf[p, r, j, sl] * SCALE
            return carry
        lax.fori_loop(0, GRP // 8, r_body, 0)
        pltpu.async_copy(buf.at[p], out_hbm.at[st, wid], osems[p])

    fire(0, 0)

    def pair_body(u, carry):
        for q in range(2):
            st = 2 * u + q
            @pl.when(st + 1 < N_STAGE)
            def _fire_next():
                @pl.when(st - 1 >= 0)
                def _wait_store():
                    store_drain(st - 1, 1 - q)
                fire(st + 1, 1 - q)
            drain(q)
            scale_store(st, q)
        return carry
    lax.fori_loop(0, N_STAGE // 2, pair_body, 0)
    store_drain(N_STAGE - 2, 0)
    store_drain(N_STAGE - 1, 1)


def kernel(x, W):
    idx = jnp.transpose(x.reshape(BATCH, HIST)).astype(jnp.int32)
    table = W.reshape(VOCAB // 8, 8, D_MODEL)
    out = _gather_scale(idx, table)
    return jnp.transpose(out.reshape(HIST, BATCH, D_MODEL), (1, 0, 2))


# speedup vs baseline: 1.5728x; 1.0008x over previous
import functools

import jax
import jax.numpy as jnp
from jax import lax
from jax.experimental import pallas as pl
from jax.experimental.pallas import tpu as pltpu
from jax.experimental.pallas import tpu_sc as plsc

D_MODEL = 64
VOCAB = 1000000
BATCH = 4096
HIST = 50

NC = 2
NS = 16
NW = NC * NS

GRP = BATCH // NW
N_STAGE = HIST

SCALE = 8.0


def _mesh():
    return plsc.VectorSubcoreMesh(core_axis_name="c", subcore_axis_name="s")


@functools.partial(
    pl.kernel,
    mesh=_mesh(),
    out_type=jax.ShapeDtypeStruct((HIST, NW, GRP // 8, 8, D_MODEL),
                                  jnp.float32),
    scratch_types=[
        pltpu.VMEM((N_STAGE, GRP), jnp.int32),
        pltpu.VMEM((2, GRP // 8, 8, D_MODEL), jnp.float32),
        pltpu.SemaphoreType.DMA,
        pltpu.SemaphoreType.DMA,
        pltpu.SemaphoreType.DMA,
        pltpu.SemaphoreType.DMA,
    ],
    compiler_params=pltpu.CompilerParams(needs_layout_passes=False),
)
def _gather_scale(idx_hbm, table_hbm, out_hbm, idx_v, buf, sem0, sem1,
                  osem0, osem1):
    wid = lax.axis_index("s") * NC + lax.axis_index("c")
    b0 = pl.multiple_of(wid * GRP, GRP)
    sems = (sem0, sem1)
    osems = (osem0, osem1)
    pltpu.sync_copy(idx_hbm.at[:, pl.ds(b0, GRP)], idx_v)

    def store_drain(st, p):
        pltpu.make_async_copy(
            buf.at[p], out_hbm.at[st, wid], osems[p]).wait()

    def fire(st, p):
        def c_body(c, carry):
            chunk = idx_v[st, pl.ds(c * 16, 16)]
            hi = lax.shift_right_logical(chunk, 3)
            lo = lax.bitwise_and(chunk, 7)
            for i in range(16):
                pltpu.async_copy(
                    table_hbm.at[hi[i], lo[i]],
                    buf.at[p, 2 * c + i // 8, i % 8],
                    sems[p])
            return carry
        lax.fori_loop(0, GRP // 16, c_body, 0)

    def drain(p):
        pltpu.make_async_copy(
            table_hbm.at[pl.ds(0, GRP // 8)], buf.at[p], sems[p]).wait()

    def scale_store(st, p):
        def r_body(r, carry):
            for j in range(8):
                for q in range(D_MODEL // 16):
                    sl = pl.ds(q * 16, 16)
                    buf[p, r, j, sl] = buf[p, r, j, sl] * SCALE
            return carry
        lax.fori_loop(0, GRP // 8, r_body, 0)
        pltpu.async_copy(buf.at[p], out_hbm.at[st, wid], osems[p])

    fire(0, 0)

    def pair_body(u, carry):
        for q in range(2):
            st = 2 * u + q
            @pl.when(st + 1 < N_STAGE)
            def _fire_next():
                @pl.when(st - 1 >= 0)
                def _wait_store():
                    store_drain(st - 1, 1 - q)
                fire(st + 1, 1 - q)
            drain(q)
            scale_store(st, q)
        return carry
    lax.fori_loop(0, N_STAGE // 2, pair_body, 0)
    store_drain(N_STAGE - 2, 0)
    store_drain(N_STAGE - 1, 1)


def kernel(x, W):
    idx = jnp.transpose(x.reshape(BATCH, HIST)).astype(jnp.int32)
    table = W.reshape(VOCAB // 8, 8, D_MODEL)
    out = _gather_scale(idx, table)
    return jnp.transpose(out.reshape(HIST, BATCH, D_MODEL), (1, 0, 2))
